# baseline (speedup 1.0000x reference)
import functools
import math

import jax
import jax.numpy as jnp
from jax import lax
from jax.experimental import pallas as pl
from jax.experimental.pallas import tpu as pltpu
from jax.experimental.pallas import tpu_sc as plsc

D = 32
SCALE = math.sqrt(32.0)
NC, NS = 2, 16
NW = NC * NS
SEQ = 200
ROWS = 4096
RT = ROWS // 128
UNITS = SEQ * RT
UPW = UNITS // NW

_mesh = plsc.VectorSubcoreMesh(
    core_axis_name="c", subcore_axis_name="s", num_cores=NC, num_subcores=NS
)


def _emb_body(table_hbm, tok_hbm, out_hbm, idx_v, rows_v, tiles_v, sem):
    wid = lax.axis_index("s") * NC + lax.axis_index("c")
    lane = lax.iota(jnp.int32, 16)

    def unit(u, carry):
        uid = wid * UPW + u
        c = uid // RT
        rt = uid % RT
        pltpu.sync_copy(tok_hbm.at[c, pl.ds(rt * 128, 128)], idx_v)
        pltpu.async_copy(table_hbm.at[idx_v], rows_v, sem).wait()

        def scat(r, carry2):
            for h in range(2):
                f = lane + (16 * h)
                vals = rows_v[r, pl.ds(16 * h, 16)] * SCALE
                plsc.store_scatter(
                    tiles_v,
                    [
                        lax.shift_right_logical(f, 3),
                        lax.bitwise_and(f, 7),
                        jnp.full((16,), r, jnp.int32),
                    ],
                    vals,
                )
            return carry2

        lax.fori_loop(0, 128, scat, 0)
        for fb in range(4):
            pltpu.sync_copy(tiles_v.at[fb], out_hbm.at[c, fb, rt])
        return carry

    lax.fori_loop(0, UPW, unit, 0)


_emb_lookup = pl.kernel(
    _emb_body,
    out_type=jax.ShapeDtypeStruct((SEQ, 4, RT, 8, 128), jnp.float32),
    mesh=_mesh,
    compiler_params=pltpu.CompilerParams(
        use_tc_tiling_on_sc=False, needs_layout_passes=False
    ),
    scratch_types=[
        pltpu.VMEM((128,), jnp.int32),
        pltpu.VMEM((128, D), jnp.float32),
        pltpu.VMEM((4, 8, 128), jnp.float32),
        pltpu.SemaphoreType.DMA,
    ],
)


@jax.jit
def kernel(tokens, table):
    tok_t = tokens.T.astype(jnp.int32)
    out5 = _emb_lookup(table, tok_t)
    return out5.transpose(2, 4, 0, 1, 3).reshape(ROWS, SEQ, D)


# speedup vs baseline: 1.1443x; 1.1443x over previous
import functools
import math

import jax
import jax.numpy as jnp
from jax import lax
from jax.experimental import pallas as pl
from jax.experimental.pallas import tpu as pltpu
from jax.experimental.pallas import tpu_sc as plsc

D = 32
SCALE = math.sqrt(32.0)
NC, NS = 2, 16
NW = NC * NS
SEQ = 200
ROWS = 4096
RT = ROWS // 128
GRP = 8
NGRP = SEQ // GRP

_mesh = plsc.VectorSubcoreMesh(
    core_axis_name="c", subcore_axis_name="s", num_cores=NC, num_subcores=NS
)


def _emb_body(table_hbm, tok_hbm, out_hbm, idx2_v, rows_v, tiles_v, sem):
    w = lax.axis_index("s") * NC + lax.axis_index("c")
    lane = lax.iota(jnp.int32, 16)
    ridx = [lane + (16 * k) for k in range(8)]

    pltpu.sync_copy(tok_hbm.at[:, pl.ds(w * 128, 128)], idx2_v)

    def group(g, carry):
        c0 = g * GRP
        cps = [
            pltpu.async_copy(
                table_hbm.at[idx2_v.at[c0 + b]], rows_v.at[b], sem
            )
            for b in range(GRP)
        ]
        for b in range(GRP):
            cps[b].wait()
            rb = rows_v.at[b]

            def assemble(f, carry2):
                colv = jnp.full((16,), f, jnp.int32)
                fb = lax.shift_right_logical(f, 3)
                fi = lax.bitwise_and(f, 7)
                for k in range(8):
                    vals = plsc.load_gather(rb, [ridx[k], colv]) * SCALE
                    tiles_v[b, fb, fi, pl.ds(16 * k, 16)] = vals
                return carry2

            lax.fori_loop(0, D, assemble, 0)
        pltpu.sync_copy(tiles_v, out_hbm.at[pl.ds(c0, GRP), :, w])
        return carry

    lax.fori_loop(0, NGRP, group, 0)


_emb_lookup = pl.kernel(
    _emb_body,
    out_type=jax.ShapeDtypeStruct((SEQ, 4, RT, 8, 128), jnp.float32),
    mesh=_mesh,
    compiler_params=pltpu.CompilerParams(
        use_tc_tiling_on_sc=False, needs_layout_passes=False
    ),
    scratch_types=[
        pltpu.VMEM((SEQ, 128), jnp.int32),
        pltpu.VMEM((GRP, 128, D), jnp.float32),
        pltpu.VMEM((GRP, 4, 8, 128), jnp.float32),
        pltpu.SemaphoreType.DMA,
    ],
)


@jax.jit
def kernel(tokens, table):
    tok_t = tokens.T.astype(jnp.int32)
    out5 = _emb_lookup(table, tok_t)
    return out5.transpose(2, 4, 0, 1, 3).reshape(ROWS, SEQ, D)


# speedup vs baseline: 1.7524x; 1.5314x over previous
import functools
import math

import jax
import jax.numpy as jnp
from jax import lax
from jax.experimental import pallas as pl
from jax.experimental.pallas import tpu as pltpu
from jax.experimental.pallas import tpu_sc as plsc

D = 32
SCALE = math.sqrt(32.0)
NC, NS = 2, 16
NW = NC * NS
SEQ = 200
ROWS = 4096
RT = ROWS // 128
GRP = 8
NGRP = SEQ // GRP

_mesh = plsc.VectorSubcoreMesh(
    core_axis_name="c", subcore_axis_name="s", num_cores=NC, num_subcores=NS
)


def _emb_body(table_hbm, tok_hbm, out_hbm, idx2_v, rows_v, tiles_v, sem):
    w = lax.axis_index("s") * NC + lax.axis_index("c")
    lane = lax.iota(jnp.int32, 16)
    fbs = [lax.shift_right_logical(lane + 16 * h, 3) for h in range(2)]
    fis = [lax.bitwise_and(lane + 16 * h, 7) for h in range(2)]

    pltpu.sync_copy(tok_hbm.at[:, pl.ds(w * 128, 128)], idx2_v)

    def group(g, carry):
        c0 = g * GRP
        cps = [
            pltpu.async_copy(
                table_hbm.at[idx2_v.at[c0 + b]], rows_v.at[b], sem
            )
            for b in range(GRP)
        ]
        for b in range(GRP):
            cps[b].wait()
            tb = tiles_v.at[b]

            def assemble(r4, carry2):
                for j in range(4):
                    r = r4 * 4 + j
                    rv = jnp.full((16,), r, jnp.int32)
                    for h in range(2):
                        vals = rows_v[b, r, pl.ds(16 * h, 16)] * SCALE
                        plsc.store_scatter(tb, [fbs[h], fis[h], rv], vals)
                return carry2

            lax.fori_loop(0, 32, assemble, 0)
        pltpu.sync_copy(
            tiles_v.at[:, :, :, pl.ds(0, 128)],
            out_hbm.at[pl.ds(c0, GRP), :, w],
        )
        return carry

    lax.fori_loop(0, NGRP, group, 0)


_emb_lookup = pl.kernel(
    _emb_body,
    out_type=jax.ShapeDtypeStruct((SEQ, 4, RT, 8, 128), jnp.float32),
    mesh=_mesh,
    compiler_params=pltpu.CompilerParams(
        use_tc_tiling_on_sc=False, needs_layout_passes=False
    ),
    scratch_types=[
        pltpu.VMEM((SEQ, 128), jnp.int32),
        pltpu.VMEM((GRP, 128, D), jnp.float32),
        pltpu.VMEM((GRP, 4, 8, 133), jnp.float32),
        pltpu.SemaphoreType.DMA,
    ],
)


@jax.jit
def kernel(tokens, table):
    tok_t = tokens.T.astype(jnp.int32)
    out5 = _emb_lookup(table, tok_t)
    return out5.transpose(2, 4, 0, 1, 3).reshape(ROWS, SEQ, D)
